# Initial kernel scaffold; baseline (speedup 1.0000x reference)
import functools

import jax
import jax.numpy as jnp
from jax import lax
from jax.experimental import pallas as pl
from jax.experimental.pallas import tpu as pltpu
from jax.experimental.pallas import tpu_sc as plsc

B, L = 4096, 200
CHAR_D, SEG_D, POS_D, BICHAR_D = 64, 16, 32, 64
D_TOT = CHAR_D + SEG_D + POS_D + BICHAR_D
N = B * L

NC, NS = 2, 16
NW = NC * NS
TOK_PER_W = N // NW
CHUNK = 128
CHUNKS_PER_W = TOK_PER_W // CHUNK

_mesh = plsc.VectorSubcoreMesh(core_axis_name="c", subcore_axis_name="s")


@functools.partial(
    pl.kernel,
    mesh=_mesh,
    out_type=jax.ShapeDtypeStruct((N, D_TOT), jnp.float32),
    scratch_types=[
        pltpu.VMEM((4, CHUNK), jnp.int32),
        pltpu.VMEM((CHUNK, CHAR_D), jnp.float32),
        pltpu.VMEM((CHUNK, SEG_D), jnp.float32),
        pltpu.VMEM((CHUNK, POS_D), jnp.float32),
        pltpu.VMEM((CHUNK, BICHAR_D), jnp.float32),
        pltpu.SemaphoreType.DMA,
    ],
)
def _mix_embed(idx_hbm, char_hbm, seg_hbm, pos_hbm, bichar_hbm, out_hbm,
               idx_v, char_v, seg_v, pos_v, bichar_v, sem):
    wid = lax.axis_index("s") * NC + lax.axis_index("c")
    first_chunk = wid * CHUNKS_PER_W

    def body(i, carry):
        chunk_id = first_chunk + i
        base = chunk_id * CHUNK
        pltpu.sync_copy(idx_hbm.at[chunk_id], idx_v)
        c1 = pltpu.async_copy(char_hbm.at[idx_v.at[0]], char_v, sem)
        c2 = pltpu.async_copy(seg_hbm.at[idx_v.at[1]], seg_v, sem)
        c3 = pltpu.async_copy(pos_hbm.at[idx_v.at[2]], pos_v, sem)
        c4 = pltpu.async_copy(bichar_hbm.at[idx_v.at[3]], bichar_v, sem)
        c1.wait()
        c2.wait()
        c3.wait()
        c4.wait()
        pltpu.sync_copy(char_v, out_hbm.at[pl.ds(base, CHUNK), pl.ds(0, CHAR_D)])
        pltpu.sync_copy(seg_v, out_hbm.at[pl.ds(base, CHUNK), pl.ds(CHAR_D, SEG_D)])
        pltpu.sync_copy(pos_v, out_hbm.at[pl.ds(base, CHUNK), pl.ds(CHAR_D + SEG_D, POS_D)])
        pltpu.sync_copy(bichar_v, out_hbm.at[pl.ds(base, CHUNK), pl.ds(CHAR_D + SEG_D + POS_D, BICHAR_D)])
        return carry

    lax.fori_loop(0, CHUNKS_PER_W, body, 0)


def kernel(pad_chars, pad_bichars, pad_segs, pad_poss, char_table, bichar_table, seg_table, pos_table):
    idx = jnp.stack(
        [pad_chars.reshape(-1), pad_segs.reshape(-1), pad_poss.reshape(-1), pad_bichars.reshape(-1)],
        axis=0,
    ).astype(jnp.int32)
    idx = idx.reshape(4, N // CHUNK, CHUNK).transpose(1, 0, 2)
    out = _mix_embed(idx, char_table, seg_table, pos_table, bichar_table)
    return out.reshape(B, L, D_TOT)


# speedup vs baseline: 1.4498x; 1.4498x over previous
import functools

import jax
import jax.numpy as jnp
from jax import lax
from jax.experimental import pallas as pl
from jax.experimental.pallas import tpu as pltpu
from jax.experimental.pallas import tpu_sc as plsc

B, L = 4096, 200
CHAR_D, SEG_D, POS_D, BICHAR_D = 64, 16, 32, 64
D_TOT = CHAR_D + SEG_D + POS_D + BICHAR_D
N = B * L

NC, NS = 2, 16
NW = NC * NS
TOK_PER_W = N // NW
CHUNK = 128
CHUNKS_PER_W = TOK_PER_W // CHUNK

_mesh = plsc.VectorSubcoreMesh(core_axis_name="c", subcore_axis_name="s")


@functools.partial(
    pl.kernel,
    mesh=_mesh,
    out_type=jax.ShapeDtypeStruct((N, D_TOT), jnp.float32),
    scratch_types=[
        pltpu.VMEM((4, CHUNK), jnp.int32),
        pltpu.VMEM((CHUNK, CHAR_D), jnp.float32),
        pltpu.VMEM((CHUNK, SEG_D), jnp.float32),
        pltpu.VMEM((CHUNK, POS_D), jnp.float32),
        pltpu.VMEM((CHUNK, BICHAR_D), jnp.float32),
        pltpu.SemaphoreType.DMA,
    ],
    compiler_params=pltpu.CompilerParams(use_tc_tiling_on_sc=False),
)
def _mix_embed(idx_hbm, char_hbm, seg_hbm, pos_hbm, bichar_hbm, out_hbm,
               idx_v, char_v, seg_v, pos_v, bichar_v, sem):
    wid = lax.axis_index("s") * NC + lax.axis_index("c")
    first_chunk = wid * CHUNKS_PER_W

    def body(i, carry):
        chunk_id = first_chunk + i
        base = chunk_id * CHUNK
        pltpu.sync_copy(idx_hbm.at[chunk_id], idx_v)
        c1 = pltpu.async_copy(char_hbm.at[idx_v.at[0]], char_v, sem)
        c2 = pltpu.async_copy(seg_hbm.at[idx_v.at[1]], seg_v, sem)
        c3 = pltpu.async_copy(pos_hbm.at[idx_v.at[2]], pos_v, sem)
        c4 = pltpu.async_copy(bichar_hbm.at[idx_v.at[3]], bichar_v, sem)
        c1.wait()
        c2.wait()
        c3.wait()
        c4.wait()
        pltpu.sync_copy(char_v, out_hbm.at[pl.ds(base, CHUNK), pl.ds(0, CHAR_D)])
        pltpu.sync_copy(seg_v, out_hbm.at[pl.ds(base, CHUNK), pl.ds(CHAR_D, SEG_D)])
        pltpu.sync_copy(pos_v, out_hbm.at[pl.ds(base, CHUNK), pl.ds(CHAR_D + SEG_D, POS_D)])
        pltpu.sync_copy(bichar_v, out_hbm.at[pl.ds(base, CHUNK), pl.ds(CHAR_D + SEG_D + POS_D, BICHAR_D)])
        return carry

    lax.fori_loop(0, CHUNKS_PER_W, body, 0)


def kernel(pad_chars, pad_bichars, pad_segs, pad_poss, char_table, bichar_table, seg_table, pos_table):
    idx = jnp.stack(
        [pad_chars.reshape(-1), pad_segs.reshape(-1), pad_poss.reshape(-1), pad_bichars.reshape(-1)],
        axis=0,
    ).astype(jnp.int32)
    idx = idx.reshape(4, N // CHUNK, CHUNK).transpose(1, 0, 2)
    out = _mix_embed(idx, char_table, seg_table, pos_table, bichar_table)
    return out.reshape(B, L, D_TOT)
